# baseline (speedup 1.0000x reference)
import functools

import jax
import jax.numpy as jnp
from jax import lax
from jax.experimental import pallas as pl
from jax.experimental.pallas import tpu as pltpu
from jax.experimental.pallas import tpu_sc as plsc

_B = 1024
_H = 64
_NC = 2
_NS = 16
_NW = _NC * _NS
_BPW = _B // _NW

_BC = 2048
_BN = 4096
_NBUF = 3

_sc_mesh = plsc.VectorSubcoreMesh(core_axis_name="c", subcore_axis_name="s")


@functools.partial(
    pl.kernel,
    mesh=_sc_mesh,
    out_type=jax.ShapeDtypeStruct((_B, 2 * _H), jnp.float32),
    scratch_types=[
        pltpu.VMEM((_BPW,), jnp.int32),
        pltpu.VMEM((_BPW, 2 * _H), jnp.float32),
        pltpu.SemaphoreType.DMA,
    ],
)
def _sc_gather(table_hbm, idx_hbm, out_hbm, idx_v, rows_v, sem):
    wid = lax.axis_index("s") * _NC + lax.axis_index("c")
    base = wid * _BPW
    pltpu.sync_copy(idx_hbm.at[pl.ds(base, _BPW)], idx_v)
    pltpu.async_copy(table_hbm.at[idx_v], rows_v, sem).wait()
    pltpu.sync_copy(rows_v, out_hbm.at[pl.ds(base, _BPW)])


def _tr_body(et_ref, o_ref):
    t = jnp.transpose(et_ref[...])
    o_ref[...] = jnp.concatenate([t, t], axis=1)


def _make_mm_body(ng, v_tail):
    def _mm_body(g_ref, wt_ref, o_hbm, ht_ref, obuf, sems):
        i = pl.program_id(0)
        slot = lax.rem(i, _NBUF)

        @pl.when(i == 0)
        def _():
            h = g_ref[...][:, :_H]
            ht_ref[...] = jnp.transpose(h).astype(jnp.bfloat16)

        @pl.when(i >= _NBUF)
        def _():
            pltpu.make_async_copy(
                obuf.at[slot],
                o_hbm.at[pl.ds((i - _NBUF) * _BN, _BN)],
                sems.at[slot],
            ).wait()

        obuf[slot] = lax.dot_general(
            wt_ref[...].astype(jnp.bfloat16),
            ht_ref[...],
            dimension_numbers=(((0,), (0,)), ((), ())),
            preferred_element_type=jnp.float32,
        )

        @pl.when(i < ng - 1)
        def _():
            pltpu.make_async_copy(
                obuf.at[slot],
                o_hbm.at[pl.ds(i * _BN, _BN)],
                sems.at[slot],
            ).start()

        @pl.when(i == ng - 1)
        def _():
            pltpu.make_async_copy(
                obuf.at[slot, pl.ds(0, v_tail)],
                o_hbm.at[pl.ds(i * _BN, v_tail)],
                sems.at[slot],
            ).start()
            for k in range(_NBUF - 1):
                j = ng - _NBUF + k
                pltpu.make_async_copy(
                    obuf.at[j % _NBUF],
                    o_hbm.at[pl.ds(j * _BN, _BN)],
                    sems.at[j % _NBUF],
                ).wait()
            pltpu.make_async_copy(
                obuf.at[slot, pl.ds(0, v_tail)],
                o_hbm.at[pl.ds(i * _BN, v_tail)],
                sems.at[slot],
            ).wait()

    return _mm_body


def kernel(x, emb, W):
    xi = x.astype(jnp.int32)
    V = W.shape[0]
    et = emb.T
    wt = W.T

    table = pl.pallas_call(
        _tr_body,
        grid=(pl.cdiv(V, _BC),),
        in_specs=[pl.BlockSpec((_H, _BC), lambda i: (0, i))],
        out_specs=pl.BlockSpec((_BC, 2 * _H), lambda i: (i, 0)),
        out_shape=jax.ShapeDtypeStruct((V, 2 * _H), jnp.float32),
        compiler_params=pltpu.CompilerParams(
            dimension_semantics=("arbitrary",),
        ),
    )(et)

    g = _sc_gather(table, xi)

    ng = pl.cdiv(V, _BN)
    v_tail = V - (ng - 1) * _BN
    lt = pl.pallas_call(
        _make_mm_body(ng, v_tail),
        grid=(ng,),
        in_specs=[
            pl.BlockSpec((_B, 2 * _H), lambda i: (0, 0)),
            pl.BlockSpec((_H, _BN), lambda i: (0, i)),
        ],
        out_specs=pl.BlockSpec(memory_space=pl.ANY),
        out_shape=jax.ShapeDtypeStruct((V, _B), jnp.float32),
        scratch_shapes=[
            pltpu.VMEM((_H, _B), jnp.bfloat16),
            pltpu.VMEM((_NBUF, _BN, _B), jnp.float32),
            pltpu.SemaphoreType.DMA((_NBUF,)),
        ],
        compiler_params=pltpu.CompilerParams(
            dimension_semantics=("arbitrary",),
        ),
    )(g, wt)
    return lt.T


# speedup vs baseline: 1.0980x; 1.0980x over previous
import functools

import jax
import jax.numpy as jnp
from jax import lax
from jax.experimental import pallas as pl
from jax.experimental.pallas import tpu as pltpu
from jax.experimental.pallas import tpu_sc as plsc

_B = 1024
_H = 64
_NC = 2
_NS = 16
_NW = _NC * _NS
_BPW = _B // _NW

_BC = 8192
_BN = 4096
_NBUF = 3

_sc_mesh = plsc.VectorSubcoreMesh(core_axis_name="c", subcore_axis_name="s")


@functools.partial(
    pl.kernel,
    mesh=_sc_mesh,
    out_type=jax.ShapeDtypeStruct((_B, 2 * _H), jnp.float32),
    scratch_types=[
        pltpu.VMEM((_BPW,), jnp.int32),
        pltpu.VMEM((_BPW, 2 * _H), jnp.float32),
        pltpu.SemaphoreType.DMA,
    ],
)
def _sc_gather(table_hbm, idx_hbm, out_hbm, idx_v, rows_v, sem):
    wid = lax.axis_index("s") * _NC + lax.axis_index("c")
    base = wid * _BPW
    pltpu.sync_copy(idx_hbm.at[pl.ds(base, _BPW)], idx_v)
    pltpu.async_copy(table_hbm.at[idx_v], rows_v, sem).wait()
    pltpu.sync_copy(rows_v, out_hbm.at[pl.ds(base, _BPW)])


def _tr_body(et_ref, o_ref):
    t = jnp.transpose(et_ref[...])
    o_ref[...] = jnp.concatenate([t, t], axis=1)


def _make_mm_body(ng, v_tail):
    def _mm_body(g_ref, wt_ref, o_hbm, ht_ref, obuf, sems):
        i = pl.program_id(0)
        slot = lax.rem(i, _NBUF)

        @pl.when(i == 0)
        def _():
            h = g_ref[...][:, :_H]
            ht_ref[...] = jnp.transpose(h).astype(jnp.bfloat16)

        @pl.when(i >= _NBUF)
        def _():
            pltpu.make_async_copy(
                obuf.at[slot],
                o_hbm.at[pl.ds((i - _NBUF) * _BN, _BN)],
                sems.at[slot],
            ).wait()

        obuf[slot] = lax.dot_general(
            wt_ref[...].astype(jnp.bfloat16),
            ht_ref[...],
            dimension_numbers=(((0,), (0,)), ((), ())),
            preferred_element_type=jnp.float32,
        )

        @pl.when(i < ng - 1)
        def _():
            pltpu.make_async_copy(
                obuf.at[slot],
                o_hbm.at[pl.ds(i * _BN, _BN)],
                sems.at[slot],
            ).start()

        @pl.when(i == ng - 1)
        def _():
            pltpu.make_async_copy(
                obuf.at[slot, pl.ds(0, v_tail)],
                o_hbm.at[pl.ds(i * _BN, v_tail)],
                sems.at[slot],
            ).start()
            for k in range(_NBUF - 1):
                j = ng - _NBUF + k
                pltpu.make_async_copy(
                    obuf.at[j % _NBUF],
                    o_hbm.at[pl.ds(j * _BN, _BN)],
                    sems.at[j % _NBUF],
                ).wait()
            pltpu.make_async_copy(
                obuf.at[slot, pl.ds(0, v_tail)],
                o_hbm.at[pl.ds(i * _BN, v_tail)],
                sems.at[slot],
            ).wait()

    return _mm_body


def kernel(x, emb, W):
    xi = x.astype(jnp.int32)
    V = W.shape[0]
    et = emb.T
    wt = W.T

    table = pl.pallas_call(
        _tr_body,
        grid=(pl.cdiv(V, _BC),),
        in_specs=[pl.BlockSpec((_H, _BC), lambda i: (0, i))],
        out_specs=pl.BlockSpec((_BC, 2 * _H), lambda i: (i, 0)),
        out_shape=jax.ShapeDtypeStruct((V, 2 * _H), jnp.float32),
        compiler_params=pltpu.CompilerParams(
            dimension_semantics=("arbitrary",),
        ),
    )(et)

    g = _sc_gather(table, xi)

    ng = pl.cdiv(V, _BN)
    v_tail = V - (ng - 1) * _BN
    lt = pl.pallas_call(
        _make_mm_body(ng, v_tail),
        grid=(ng,),
        in_specs=[
            pl.BlockSpec((_B, 2 * _H), lambda i: (0, 0)),
            pl.BlockSpec((_H, _BN), lambda i: (0, i)),
        ],
        out_specs=pl.BlockSpec(memory_space=pl.ANY),
        out_shape=jax.ShapeDtypeStruct((V, _B), jnp.float32),
        scratch_shapes=[
            pltpu.VMEM((_H, _B), jnp.bfloat16),
            pltpu.VMEM((_NBUF, _BN, _B), jnp.float32),
            pltpu.SemaphoreType.DMA((_NBUF,)),
        ],
        compiler_params=pltpu.CompilerParams(
            dimension_semantics=("arbitrary",),
        ),
    )(g, wt)
    return lt.T
